# Initial kernel scaffold; baseline (speedup 1.0000x reference)
import jax
import jax.numpy as jnp
from jax.experimental import pallas as pl


def kernel(source_feat, source_softmax, source_confidence, target_feat, target_softmax, target_confidence):
    raise NotImplementedError("write your pallas kernel here")


# speedup vs baseline: 3.4361x; 3.4361x over previous
import functools

import jax
import jax.numpy as jnp
from jax import lax
from jax.experimental import pallas as pl

K = 19
_PREC = lax.Precision.HIGHEST


def _labels_of(sm):
    best = sm[0:1, :]
    arg = jnp.zeros(best.shape, dtype=jnp.int32)
    for k in range(1, K):
        v = sm[k : k + 1, :]
        gt = v > best
        best = jnp.where(gt, v, best)
        arg = jnp.where(gt, k, arg)
    return arg


def _pass1_kernel(sf_ref, tf_ref, ssm_ref, tsm_ref, sw_ref, tw_ref,
                  num_ref, aux_ref, slab_ref, tlab_ref):
    i = pl.program_id(0)
    nt = sf_ref.shape[2]

    s_arg = _labels_of(ssm_ref[0])
    t_arg = _labels_of(tsm_ref[0])
    slab_ref[0] = s_arg
    tlab_ref[0] = t_arg

    iota = lax.broadcasted_iota(jnp.int32, (K, nt), 0)
    s_oh = (iota == s_arg).astype(jnp.float32)
    t_oh = (iota == t_arg).astype(jnp.float32)
    ms = s_oh * sw_ref[0]
    mt = t_oh * tw_ref[0]

    dn = (((1,), (1,)), ((), ()))
    contrib = lax.dot_general(sf_ref[0], ms, dn,
                              preferred_element_type=jnp.float32,
                              precision=_PREC)
    contrib += lax.dot_general(tf_ref[0], mt, dn,
                               preferred_element_type=jnp.float32,
                               precision=_PREC)

    den_s = jnp.sum(ms, axis=1, keepdims=True)
    den_t = jnp.sum(mt, axis=1, keepdims=True)
    cnt_s = jnp.sum(s_oh, axis=1, keepdims=True)
    cnt_t = jnp.sum(t_oh, axis=1, keepdims=True)
    aux = jnp.concatenate([den_s, den_t, cnt_s, cnt_t], axis=1)

    @pl.when(i == 0)
    def _():
        num_ref[...] = jnp.zeros_like(num_ref)
        aux_ref[...] = jnp.zeros_like(aux_ref)

    num_ref[...] += contrib
    aux_ref[...] += aux


def _pass2_kernel(cen_ref, sf_ref, tf_ref, slab_ref, tlab_ref,
                  sd_ref, td_ref):
    i = pl.program_id(0)
    nt = sf_ref.shape[2]
    cen = cen_ref[...]
    c2 = jnp.sum(cen * cen, axis=1, keepdims=True)

    def seg_d(f, lab):
        fc = lax.dot_general(cen, f, (((1,), (0,)), ((), ())),
                             preferred_element_type=jnp.float32,
                             precision=_PREC)
        f2 = jnp.sum(f * f, axis=0, keepdims=True)
        d2 = jnp.maximum(f2 + c2 - 2.0 * fc, 0.0)
        dist = jnp.sqrt(d2 + 1e-12) * (1.0 / f.shape[0])
        oh = (lax.broadcasted_iota(jnp.int32, (K, nt), 0) == lab)
        oh = oh.astype(jnp.float32)
        return lax.dot_general(dist, oh, (((1,), (1,)), ((), ())),
                               preferred_element_type=jnp.float32,
                               precision=_PREC)

    contrib_s = seg_d(sf_ref[0], slab_ref[0])
    contrib_t = seg_d(tf_ref[0], tlab_ref[0])

    @pl.when(i == 0)
    def _():
        sd_ref[...] = jnp.zeros_like(sd_ref)
        td_ref[...] = jnp.zeros_like(td_ref)

    sd_ref[...] += contrib_s
    td_ref[...] += contrib_t


@functools.partial(jax.jit, static_argnames=("nt",))
def _run(sf, ssm, sw, tf, tsm, tw, nt=8192):
    B, C, N = sf.shape
    T = N // nt
    steps = B * T

    feat_spec = pl.BlockSpec((1, C, nt), lambda i: (i // T, 0, i % T))
    sm_spec = pl.BlockSpec((1, K, nt), lambda i: (i // T, 0, i % T))
    w_spec = pl.BlockSpec((1, 1, nt), lambda i: (i // T, 0, i % T))
    lab_spec = pl.BlockSpec((1, 1, nt), lambda i: (i // T, 0, i % T))

    num, aux, slab, tlab = pl.pallas_call(
        _pass1_kernel,
        grid=(steps,),
        in_specs=[feat_spec, feat_spec, sm_spec, sm_spec, w_spec, w_spec],
        out_specs=[
            pl.BlockSpec((C, K), lambda i: (0, 0)),
            pl.BlockSpec((K, 4), lambda i: (0, 0)),
            lab_spec,
            lab_spec,
        ],
        out_shape=[
            jax.ShapeDtypeStruct((C, K), jnp.float32),
            jax.ShapeDtypeStruct((K, 4), jnp.float32),
            jax.ShapeDtypeStruct((B, 1, N), jnp.int32),
            jax.ShapeDtypeStruct((B, 1, N), jnp.int32),
        ],
    )(sf, tf, ssm, tsm, sw, tw)

    den = aux[:, 0] + aux[:, 1]
    cen = num.T / jnp.maximum(den, 1.0)[:, None]
    cen_valid = den > 0
    counts_s = aux[:, 2]
    counts_t = aux[:, 3]

    sd, td = pl.pallas_call(
        _pass2_kernel,
        grid=(steps,),
        in_specs=[
            pl.BlockSpec((K, C), lambda i: (0, 0)),
            feat_spec, feat_spec, lab_spec, lab_spec,
        ],
        out_specs=[
            pl.BlockSpec((K, K), lambda i: (0, 0)),
            pl.BlockSpec((K, K), lambda i: (0, 0)),
        ],
        out_shape=[
            jax.ShapeDtypeStruct((K, K), jnp.float32),
            jax.ShapeDtypeStruct((K, K), jnp.float32),
        ],
    )(cen, sf, tf, slab, tlab)

    eye = jnp.eye(K, dtype=bool)

    def terms(segd, counts):
        mean_pc = segd.T / jnp.maximum(counts, 1.0)[:, None]
        valid = (counts > 0) & cen_valid
        diag = jnp.diagonal(mean_pc)
        intra = jnp.sum(jnp.where(valid, diag, 0.0)) / jnp.maximum(
            jnp.sum(valid.astype(jnp.float32)), 1.0)
        off = jnp.where(eye | (~cen_valid)[None, :], -jnp.inf, mean_pc)
        per_i_max = jnp.max(off, axis=1)
        inter_min = jnp.min(jnp.where(valid, per_i_max, jnp.inf))
        return intra, inter_min

    intra_s, inter_s = terms(sd, counts_s)
    intra_t, inter_t = terms(td, counts_t)
    return intra_s + intra_t - 0.1 * (inter_s + inter_t)


def kernel(source_feat, source_softmax, source_confidence,
           target_feat, target_softmax, target_confidence):
    B, C, h, w = source_feat.shape
    N = h * w
    sf = source_feat.reshape(B, C, N)
    tf = target_feat.reshape(B, C, N)
    ssm = source_softmax.reshape(B, K, N)
    tsm = target_softmax.reshape(B, K, N)
    sw = source_confidence.reshape(B, 1, N).astype(jnp.float32)
    tw = 1.0 - target_confidence.reshape(B, 1, N).astype(jnp.float32)
    return _run(sf, ssm, sw, tf, tsm, tw)


# speedup vs baseline: 4.7506x; 1.3826x over previous
import functools

import jax
import jax.numpy as jnp
from jax import lax
from jax.experimental import pallas as pl

K = 19
_BF = jnp.bfloat16


def _labels_of(sm):
    best = sm[0:1, :]
    arg = jnp.zeros(best.shape, dtype=jnp.int32)
    for k in range(1, K):
        v = sm[k : k + 1, :]
        gt = v > best
        best = jnp.where(gt, v, best)
        arg = jnp.where(gt, k, arg)
    return arg


def _pass1_kernel(sf_ref, tf_ref, ssm_ref, tsm_ref, sw_ref, tw_ref,
                  num_ref, aux_ref, slab_ref, tlab_ref):
    i = pl.program_id(0)
    nt = sf_ref.shape[2]

    s_arg = _labels_of(ssm_ref[0])
    t_arg = _labels_of(tsm_ref[0])
    slab_ref[0] = s_arg
    tlab_ref[0] = t_arg

    iota = lax.broadcasted_iota(jnp.int32, (K, nt), 0)
    s_oh = (iota == s_arg).astype(jnp.float32)
    t_oh = (iota == t_arg).astype(jnp.float32)
    ms = s_oh * sw_ref[0]
    mt = t_oh * tw_ref[0]

    dn = (((1,), (1,)), ((), ()))
    contrib = lax.dot_general(sf_ref[0].astype(_BF), ms.astype(_BF), dn,
                              preferred_element_type=jnp.float32)
    contrib += lax.dot_general(tf_ref[0].astype(_BF), mt.astype(_BF), dn,
                               preferred_element_type=jnp.float32)

    den_s = jnp.sum(ms, axis=1, keepdims=True)
    den_t = jnp.sum(mt, axis=1, keepdims=True)
    cnt_s = jnp.sum(s_oh, axis=1, keepdims=True)
    cnt_t = jnp.sum(t_oh, axis=1, keepdims=True)
    aux = jnp.concatenate([den_s, den_t, cnt_s, cnt_t], axis=1)

    @pl.when(i == 0)
    def _():
        num_ref[...] = jnp.zeros_like(num_ref)
        aux_ref[...] = jnp.zeros_like(aux_ref)

    num_ref[...] += contrib
    aux_ref[...] += aux


def _pass2_kernel(cen_ref, sf_ref, tf_ref, slab_ref, tlab_ref,
                  sd_ref, td_ref):
    i = pl.program_id(0)
    nt = sf_ref.shape[2]
    cen = cen_ref[...]
    c2 = jnp.sum(cen * cen, axis=1, keepdims=True)

    cen_bf = cen.astype(_BF)

    def seg_d(f, lab):
        fc = lax.dot_general(cen_bf, f.astype(_BF), (((1,), (0,)), ((), ())),
                             preferred_element_type=jnp.float32)
        f2 = jnp.sum(f * f, axis=0, keepdims=True)
        d2 = jnp.maximum(f2 + c2 - 2.0 * fc, 0.0)
        dist = jnp.sqrt(d2 + 1e-12) * (1.0 / f.shape[0])
        oh = (lax.broadcasted_iota(jnp.int32, (K, nt), 0) == lab)
        return lax.dot_general(dist.astype(_BF), oh.astype(_BF),
                               (((1,), (1,)), ((), ())),
                               preferred_element_type=jnp.float32)

    contrib_s = seg_d(sf_ref[0], slab_ref[0])
    contrib_t = seg_d(tf_ref[0], tlab_ref[0])

    @pl.when(i == 0)
    def _():
        sd_ref[...] = jnp.zeros_like(sd_ref)
        td_ref[...] = jnp.zeros_like(td_ref)

    sd_ref[...] += contrib_s
    td_ref[...] += contrib_t


@functools.partial(jax.jit, static_argnames=("nt",))
def _run(sf, ssm, sw, tf, tsm, tw, nt=8192):
    B, C, N = sf.shape
    T = N // nt
    steps = B * T

    feat_spec = pl.BlockSpec((1, C, nt), lambda i: (i // T, 0, i % T))
    sm_spec = pl.BlockSpec((1, K, nt), lambda i: (i // T, 0, i % T))
    w_spec = pl.BlockSpec((1, 1, nt), lambda i: (i // T, 0, i % T))
    lab_spec = pl.BlockSpec((1, 1, nt), lambda i: (i // T, 0, i % T))

    num, aux, slab, tlab = pl.pallas_call(
        _pass1_kernel,
        grid=(steps,),
        in_specs=[feat_spec, feat_spec, sm_spec, sm_spec, w_spec, w_spec],
        out_specs=[
            pl.BlockSpec((C, K), lambda i: (0, 0)),
            pl.BlockSpec((K, 4), lambda i: (0, 0)),
            lab_spec,
            lab_spec,
        ],
        out_shape=[
            jax.ShapeDtypeStruct((C, K), jnp.float32),
            jax.ShapeDtypeStruct((K, 4), jnp.float32),
            jax.ShapeDtypeStruct((B, 1, N), jnp.int32),
            jax.ShapeDtypeStruct((B, 1, N), jnp.int32),
        ],
    )(sf, tf, ssm, tsm, sw, tw)

    den = aux[:, 0] + aux[:, 1]
    cen = num.T / jnp.maximum(den, 1.0)[:, None]
    cen_valid = den > 0
    counts_s = aux[:, 2]
    counts_t = aux[:, 3]

    sd, td = pl.pallas_call(
        _pass2_kernel,
        grid=(steps,),
        in_specs=[
            pl.BlockSpec((K, C), lambda i: (0, 0)),
            feat_spec, feat_spec, lab_spec, lab_spec,
        ],
        out_specs=[
            pl.BlockSpec((K, K), lambda i: (0, 0)),
            pl.BlockSpec((K, K), lambda i: (0, 0)),
        ],
        out_shape=[
            jax.ShapeDtypeStruct((K, K), jnp.float32),
            jax.ShapeDtypeStruct((K, K), jnp.float32),
        ],
    )(cen, sf, tf, slab, tlab)

    eye = jnp.eye(K, dtype=bool)

    def terms(segd, counts):
        mean_pc = segd.T / jnp.maximum(counts, 1.0)[:, None]
        valid = (counts > 0) & cen_valid
        diag = jnp.diagonal(mean_pc)
        intra = jnp.sum(jnp.where(valid, diag, 0.0)) / jnp.maximum(
            jnp.sum(valid.astype(jnp.float32)), 1.0)
        off = jnp.where(eye | (~cen_valid)[None, :], -jnp.inf, mean_pc)
        per_i_max = jnp.max(off, axis=1)
        inter_min = jnp.min(jnp.where(valid, per_i_max, jnp.inf))
        return intra, inter_min

    intra_s, inter_s = terms(sd, counts_s)
    intra_t, inter_t = terms(td, counts_t)
    return intra_s + intra_t - 0.1 * (inter_s + inter_t)


def kernel(source_feat, source_softmax, source_confidence,
           target_feat, target_softmax, target_confidence):
    B, C, h, w = source_feat.shape
    N = h * w
    sf = source_feat.reshape(B, C, N)
    tf = target_feat.reshape(B, C, N)
    ssm = source_softmax.reshape(B, K, N)
    tsm = target_softmax.reshape(B, K, N)
    sw = source_confidence.reshape(B, 1, N).astype(jnp.float32)
    tw = 1.0 - target_confidence.reshape(B, 1, N).astype(jnp.float32)
    return _run(sf, ssm, sw, tf, tsm, tw)


# speedup vs baseline: 9.7839x; 2.0595x over previous
import functools

import jax
import jax.numpy as jnp
from jax import lax
from jax.experimental import pallas as pl

K = 19
_BF = jnp.bfloat16


def _labels_of(sm):
    best = sm[0]
    arg = jnp.zeros(best.shape, dtype=jnp.int32)
    for k in range(1, K):
        v = sm[k]
        gt = v > best
        best = jnp.where(gt, v, best)
        arg = jnp.where(gt, k, arg)
    return arg


def _pass1_kernel(sf_ref, tf_ref, ssm_ref, tsm_ref, sw_ref, tw_ref,
                  num_ref, aux_ref, slab_ref, tlab_ref):
    i = pl.program_id(0)
    _, hb, w = sf_ref.shape

    s_arg = _labels_of(ssm_ref[...])
    t_arg = _labels_of(tsm_ref[...])
    slab_ref[0] = s_arg
    tlab_ref[0] = t_arg

    iota = lax.broadcasted_iota(jnp.int32, (K, hb, w), 0)
    s_oh = (iota == s_arg[None]).astype(jnp.float32)
    t_oh = (iota == t_arg[None]).astype(jnp.float32)
    ms = (s_oh * sw_ref[...]).reshape(K, hb * w)
    mt = (t_oh * tw_ref[...]).reshape(K, hb * w)
    C = sf_ref.shape[0]
    fs = sf_ref[...].reshape(C, hb * w)
    ft = tf_ref[...].reshape(C, hb * w)

    dn = (((1,), (1,)), ((), ()))
    contrib = lax.dot_general(fs.astype(_BF), ms.astype(_BF), dn,
                              preferred_element_type=jnp.float32)
    contrib += lax.dot_general(ft.astype(_BF), mt.astype(_BF), dn,
                               preferred_element_type=jnp.float32)

    def colsum(x):
        return jnp.sum(x, axis=1, keepdims=True)

    aux = jnp.concatenate(
        [colsum(ms), colsum(mt),
         colsum(s_oh.reshape(K, hb * w)), colsum(t_oh.reshape(K, hb * w))],
        axis=1)

    @pl.when(i == 0)
    def _():
        num_ref[...] = jnp.zeros_like(num_ref)
        aux_ref[...] = jnp.zeros_like(aux_ref)

    num_ref[...] += contrib
    aux_ref[...] += aux


def _pass2_kernel(cen_ref, sf_ref, tf_ref, slab_ref, tlab_ref,
                  sd_ref, td_ref):
    i = pl.program_id(0)
    _, hb, w = sf_ref.shape
    cen = cen_ref[...]
    c2 = jnp.sum(cen * cen, axis=1, keepdims=True)
    cen_bf = cen.astype(_BF)

    def seg_d(f3, lab):
        C = f3.shape[0]
        f = f3.reshape(C, hb * w)
        fc = lax.dot_general(cen_bf, f.astype(_BF), (((1,), (0,)), ((), ())),
                             preferred_element_type=jnp.float32)
        f2 = jnp.sum(f * f, axis=0, keepdims=True)
        d2 = jnp.maximum(f2 + c2 - 2.0 * fc, 0.0)
        dist = jnp.sqrt(d2 + 1e-12) * (1.0 / C)
        oh = (lax.broadcasted_iota(jnp.int32, (K, hb, w), 0) == lab[None])
        oh = oh.astype(_BF).reshape(K, hb * w)
        return lax.dot_general(dist.astype(_BF), oh,
                               (((1,), (1,)), ((), ())),
                               preferred_element_type=jnp.float32)

    contrib_s = seg_d(sf_ref[...], slab_ref[0])
    contrib_t = seg_d(tf_ref[...], tlab_ref[0])

    @pl.when(i == 0)
    def _():
        sd_ref[...] = jnp.zeros_like(sd_ref)
        td_ref[...] = jnp.zeros_like(td_ref)

    sd_ref[...] += contrib_s
    td_ref[...] += contrib_t


@functools.partial(jax.jit, static_argnames=("hb",))
def _run(sf, ssm, sw, tf, tsm, tw, hb=32):
    B, C, h, w = sf.shape
    T = h // hb
    steps = B * T

    sf3 = sf.reshape(B * C, h, w)
    tf3 = tf.reshape(B * C, h, w)
    ssm3 = ssm.reshape(B * K, h, w)
    tsm3 = tsm.reshape(B * K, h, w)

    feat_spec = pl.BlockSpec((C, hb, w), lambda i: (i // T, i % T, 0))
    sm_spec = pl.BlockSpec((K, hb, w), lambda i: (i // T, i % T, 0))
    w_spec = pl.BlockSpec((1, hb, w), lambda i: (i // T, i % T, 0))
    lab_spec = pl.BlockSpec((1, hb, w), lambda i: (i // T, i % T, 0))

    num, aux, slab, tlab = pl.pallas_call(
        _pass1_kernel,
        grid=(steps,),
        in_specs=[feat_spec, feat_spec, sm_spec, sm_spec, w_spec, w_spec],
        out_specs=[
            pl.BlockSpec((C, K), lambda i: (0, 0)),
            pl.BlockSpec((K, 4), lambda i: (0, 0)),
            lab_spec,
            lab_spec,
        ],
        out_shape=[
            jax.ShapeDtypeStruct((C, K), jnp.float32),
            jax.ShapeDtypeStruct((K, 4), jnp.float32),
            jax.ShapeDtypeStruct((B, h, w), jnp.int32),
            jax.ShapeDtypeStruct((B, h, w), jnp.int32),
        ],
    )(sf3, tf3, ssm3, tsm3, sw, tw)

    den = aux[:, 0] + aux[:, 1]
    cen = num.T / jnp.maximum(den, 1.0)[:, None]
    cen_valid = den > 0
    counts_s = aux[:, 2]
    counts_t = aux[:, 3]

    sd, td = pl.pallas_call(
        _pass2_kernel,
        grid=(steps,),
        in_specs=[
            pl.BlockSpec((K, C), lambda i: (0, 0)),
            feat_spec, feat_spec, lab_spec, lab_spec,
        ],
        out_specs=[
            pl.BlockSpec((K, K), lambda i: (0, 0)),
            pl.BlockSpec((K, K), lambda i: (0, 0)),
        ],
        out_shape=[
            jax.ShapeDtypeStruct((K, K), jnp.float32),
            jax.ShapeDtypeStruct((K, K), jnp.float32),
        ],
    )(cen, sf3, tf3, slab, tlab)

    eye = jnp.eye(K, dtype=bool)

    def terms(segd, counts):
        mean_pc = segd.T / jnp.maximum(counts, 1.0)[:, None]
        valid = (counts > 0) & cen_valid
        diag = jnp.diagonal(mean_pc)
        intra = jnp.sum(jnp.where(valid, diag, 0.0)) / jnp.maximum(
            jnp.sum(valid.astype(jnp.float32)), 1.0)
        off = jnp.where(eye | (~cen_valid)[None, :], -jnp.inf, mean_pc)
        per_i_max = jnp.max(off, axis=1)
        inter_min = jnp.min(jnp.where(valid, per_i_max, jnp.inf))
        return intra, inter_min

    intra_s, inter_s = terms(sd, counts_s)
    intra_t, inter_t = terms(td, counts_t)
    return intra_s + intra_t - 0.1 * (inter_s + inter_t)


def kernel(source_feat, source_softmax, source_confidence,
           target_feat, target_softmax, target_confidence):
    sw = source_confidence.astype(jnp.float32)
    tw = 1.0 - target_confidence.astype(jnp.float32)
    return _run(source_feat, source_softmax, sw,
                target_feat, target_softmax, tw)


# speedup vs baseline: 10.6336x; 1.0868x over previous
import functools

import jax
import jax.numpy as jnp
from jax import lax
from jax.experimental import pallas as pl

K = 19
_BF = jnp.bfloat16


def _labels_of(sm):
    best = sm[0]
    arg = jnp.zeros(best.shape, dtype=jnp.int32)
    for k in range(1, K):
        v = sm[k]
        gt = v > best
        best = jnp.where(gt, v, best)
        arg = jnp.where(gt, k, arg)
    return arg


def _pass1_kernel(sf_ref, tf_ref, ssm_ref, tsm_ref, sw_ref, tw_ref,
                  num_ref, aux_ref, slab_ref, tlab_ref):
    i = pl.program_id(0)
    _, hb, w = sf_ref.shape

    s_arg = _labels_of(ssm_ref[...])
    t_arg = _labels_of(tsm_ref[...])
    slab_ref[0] = s_arg
    tlab_ref[0] = t_arg

    iota = lax.broadcasted_iota(jnp.int32, (K, hb, w), 0)
    s_oh = (iota == s_arg[None]).astype(jnp.float32)
    t_oh = (iota == t_arg[None]).astype(jnp.float32)
    ms3 = s_oh * sw_ref[...]
    mt3 = t_oh * tw_ref[...]
    C = sf_ref.shape[0]
    fs = sf_ref[...].astype(_BF).reshape(C, hb * w)
    ft = tf_ref[...].astype(_BF).reshape(C, hb * w)
    ms = ms3.astype(_BF).reshape(K, hb * w)
    mt = mt3.astype(_BF).reshape(K, hb * w)

    dn = (((1,), (1,)), ((), ()))
    contrib = lax.dot_general(fs, ms, dn,
                              preferred_element_type=jnp.float32)
    contrib += lax.dot_general(ft, mt, dn,
                               preferred_element_type=jnp.float32)

    def colsum(x):
        return jnp.sum(jnp.sum(x, axis=2), axis=1, keepdims=True)

    aux = jnp.concatenate(
        [colsum(ms3), colsum(mt3), colsum(s_oh), colsum(t_oh)],
        axis=1)

    @pl.when(i == 0)
    def _():
        num_ref[...] = jnp.zeros_like(num_ref)
        aux_ref[...] = jnp.zeros_like(aux_ref)

    num_ref[...] += contrib
    aux_ref[...] += aux


def _pass2_kernel(cen_ref, sf_ref, tf_ref, slab_ref, tlab_ref,
                  sd_ref, td_ref):
    i = pl.program_id(0)
    _, hb, w = sf_ref.shape
    cen = cen_ref[...]
    c2 = jnp.sum(cen * cen, axis=1, keepdims=True)
    cen_bf = cen.astype(_BF)

    ones_row = jnp.ones((1, cen.shape[1]), dtype=_BF)

    def seg_d(f3, lab):
        C = f3.shape[0]
        f = f3.astype(_BF).reshape(C, hb * w)
        fc = lax.dot_general(cen_bf, f, (((1,), (0,)), ((), ())),
                             preferred_element_type=jnp.float32)
        f2 = lax.dot_general(ones_row, f * f, (((1,), (0,)), ((), ())),
                             preferred_element_type=jnp.float32)
        d2 = jnp.maximum(f2 + c2 - 2.0 * fc, 0.0)
        dist = jnp.sqrt(d2 + 1e-12) * (1.0 / C)
        oh = (lax.broadcasted_iota(jnp.int32, (K, hb, w), 0) == lab[None])
        oh = oh.astype(_BF).reshape(K, hb * w)
        return lax.dot_general(dist.astype(_BF), oh,
                               (((1,), (1,)), ((), ())),
                               preferred_element_type=jnp.float32)

    contrib_s = seg_d(sf_ref[...], slab_ref[0])
    contrib_t = seg_d(tf_ref[...], tlab_ref[0])

    @pl.when(i == 0)
    def _():
        sd_ref[...] = jnp.zeros_like(sd_ref)
        td_ref[...] = jnp.zeros_like(td_ref)

    sd_ref[...] += contrib_s
    td_ref[...] += contrib_t


@functools.partial(jax.jit, static_argnames=("hb",))
def _run(sf, ssm, sw, tf, tsm, tw, hb=32):
    B, C, h, w = sf.shape
    T = h // hb
    steps = B * T

    sf3 = sf.reshape(B * C, h, w)
    tf3 = tf.reshape(B * C, h, w)
    ssm3 = ssm.reshape(B * K, h, w)
    tsm3 = tsm.reshape(B * K, h, w)

    feat_spec = pl.BlockSpec((C, hb, w), lambda i: (i // T, i % T, 0))
    sm_spec = pl.BlockSpec((K, hb, w), lambda i: (i // T, i % T, 0))
    w_spec = pl.BlockSpec((1, hb, w), lambda i: (i // T, i % T, 0))
    lab_spec = pl.BlockSpec((1, hb, w), lambda i: (i // T, i % T, 0))

    num, aux, slab, tlab = pl.pallas_call(
        _pass1_kernel,
        grid=(steps,),
        in_specs=[feat_spec, feat_spec, sm_spec, sm_spec, w_spec, w_spec],
        out_specs=[
            pl.BlockSpec((C, K), lambda i: (0, 0)),
            pl.BlockSpec((K, 4), lambda i: (0, 0)),
            lab_spec,
            lab_spec,
        ],
        out_shape=[
            jax.ShapeDtypeStruct((C, K), jnp.float32),
            jax.ShapeDtypeStruct((K, 4), jnp.float32),
            jax.ShapeDtypeStruct((B, h, w), jnp.int32),
            jax.ShapeDtypeStruct((B, h, w), jnp.int32),
        ],
    )(sf3, tf3, ssm3, tsm3, sw, tw)

    den = aux[:, 0] + aux[:, 1]
    cen = num.T / jnp.maximum(den, 1.0)[:, None]
    cen_valid = den > 0
    counts_s = aux[:, 2]
    counts_t = aux[:, 3]

    sd, td = pl.pallas_call(
        _pass2_kernel,
        grid=(steps,),
        in_specs=[
            pl.BlockSpec((K, C), lambda i: (0, 0)),
            feat_spec, feat_spec, lab_spec, lab_spec,
        ],
        out_specs=[
            pl.BlockSpec((K, K), lambda i: (0, 0)),
            pl.BlockSpec((K, K), lambda i: (0, 0)),
        ],
        out_shape=[
            jax.ShapeDtypeStruct((K, K), jnp.float32),
            jax.ShapeDtypeStruct((K, K), jnp.float32),
        ],
    )(cen, sf3, tf3, slab, tlab)

    eye = jnp.eye(K, dtype=bool)

    def terms(segd, counts):
        mean_pc = segd.T / jnp.maximum(counts, 1.0)[:, None]
        valid = (counts > 0) & cen_valid
        diag = jnp.diagonal(mean_pc)
        intra = jnp.sum(jnp.where(valid, diag, 0.0)) / jnp.maximum(
            jnp.sum(valid.astype(jnp.float32)), 1.0)
        off = jnp.where(eye | (~cen_valid)[None, :], -jnp.inf, mean_pc)
        per_i_max = jnp.max(off, axis=1)
        inter_min = jnp.min(jnp.where(valid, per_i_max, jnp.inf))
        return intra, inter_min

    intra_s, inter_s = terms(sd, counts_s)
    intra_t, inter_t = terms(td, counts_t)
    return intra_s + intra_t - 0.1 * (inter_s + inter_t)


def kernel(source_feat, source_softmax, source_confidence,
           target_feat, target_softmax, target_confidence):
    sw = source_confidence.astype(jnp.float32)
    tw = 1.0 - target_confidence.astype(jnp.float32)
    return _run(source_feat, source_softmax, sw,
                target_feat, target_softmax, tw)


# speedup vs baseline: 12.2868x; 1.1555x over previous
import functools

import jax
import jax.numpy as jnp
from jax import lax
from jax.experimental import pallas as pl
from jax.experimental.pallas import tpu as pltpu

K = 19
_BF = jnp.bfloat16


def _labels_of(sm):
    best = sm[0]
    arg = jnp.zeros(best.shape, dtype=jnp.int32)
    for k in range(1, K):
        v = sm[k]
        gt = v > best
        best = jnp.where(gt, v, best)
        arg = jnp.where(gt, k, arg)
    return arg


def _fused_kernel(T, sf_ref, tf_ref, ssm_ref, tsm_ref, sw_ref, tw_ref,
                  out_ref, num_ref, aux_ref, slab_ref, tlab_ref,
                  cen_ref, c2_ref, sd_ref, td_ref):
    p = pl.program_id(0)
    i = pl.program_id(1)
    steps = pl.num_programs(1)
    C, hb, w = sf_ref.shape
    r0 = i * hb

    @pl.when((p == 0) & (i == 0))
    def _():
        num_ref[...] = jnp.zeros_like(num_ref)
        aux_ref[...] = jnp.zeros_like(aux_ref)
        sd_ref[...] = jnp.zeros_like(sd_ref)
        td_ref[...] = jnp.zeros_like(td_ref)

    @pl.when(p == 0)
    def _pass1():
        s_arg = _labels_of(ssm_ref[...])
        t_arg = _labels_of(tsm_ref[...])
        slab_ref[pl.ds(r0, hb), :] = s_arg
        tlab_ref[pl.ds(r0, hb), :] = t_arg

        iota = lax.broadcasted_iota(jnp.int32, (K, hb, w), 0)
        s_oh = (iota == s_arg[None]).astype(jnp.float32)
        t_oh = (iota == t_arg[None]).astype(jnp.float32)
        ms3 = s_oh * sw_ref[...].astype(jnp.float32)
        mt3 = t_oh * (1.0 - tw_ref[...].astype(jnp.float32))
        fs = sf_ref[...].astype(_BF).reshape(C, hb * w)
        ft = tf_ref[...].astype(_BF).reshape(C, hb * w)
        ms = ms3.astype(_BF).reshape(K, hb * w)
        mt = mt3.astype(_BF).reshape(K, hb * w)

        dn = (((1,), (1,)), ((), ()))
        contrib = lax.dot_general(fs, ms, dn,
                                  preferred_element_type=jnp.float32)
        contrib += lax.dot_general(ft, mt, dn,
                                   preferred_element_type=jnp.float32)

        def colsum(x):
            return jnp.sum(jnp.sum(x, axis=2), axis=1, keepdims=True)

        num_ref[...] += contrib
        aux_ref[...] += jnp.concatenate(
            [colsum(ms3), colsum(mt3), colsum(s_oh), colsum(t_oh)], axis=1)

    @pl.when((p == 1) & (i == 0))
    def _centroids():
        den = aux_ref[:, 0:1] + aux_ref[:, 1:2]
        cen = num_ref[...].T / jnp.maximum(den, 1.0)
        cen_ref[...] = cen.astype(_BF)
        c2_ref[...] = jnp.sum(cen * cen, axis=1, keepdims=True)

    @pl.when(p == 1)
    def _pass2():
        cen_bf = cen_ref[...]
        c2 = c2_ref[...]
        ones_row = jnp.ones((1, C), dtype=_BF)

        def seg_d(f3, lab):
            f = f3.astype(_BF).reshape(C, hb * w)
            fc = lax.dot_general(cen_bf, f, (((1,), (0,)), ((), ())),
                                 preferred_element_type=jnp.float32)
            f2 = lax.dot_general(ones_row, f * f, (((1,), (0,)), ((), ())),
                                 preferred_element_type=jnp.float32)
            d2 = jnp.maximum(f2 + c2 - 2.0 * fc, 0.0)
            dist = jnp.sqrt(d2 + 1e-12) * (1.0 / C)
            oh = (lax.broadcasted_iota(jnp.int32, (K, hb, w), 0) == lab[None])
            oh = oh.astype(_BF).reshape(K, hb * w)
            return lax.dot_general(dist.astype(_BF), oh,
                                   (((1,), (1,)), ((), ())),
                                   preferred_element_type=jnp.float32)

        sd_ref[...] += seg_d(sf_ref[...], slab_ref[pl.ds(r0, hb), :])
        td_ref[...] += seg_d(tf_ref[...], tlab_ref[pl.ds(r0, hb), :])

    @pl.when((p == 1) & (i == steps - 1))
    def _finalize():
        den = aux_ref[:, 0:1] + aux_ref[:, 1:2]
        cen_valid = den > 0.0
        row_i = lax.broadcasted_iota(jnp.int32, (K, K), 0)
        col_j = lax.broadcasted_iota(jnp.int32, (K, K), 1)
        eye = row_i == col_j
        inf = jnp.float32(jnp.inf)

        def terms(segd, counts):
            mean_pc = segd.T / jnp.maximum(counts, 1.0)
            valid = (counts > 0.0) & cen_valid
            diag = jnp.sum(jnp.where(eye, mean_pc, 0.0), axis=1,
                           keepdims=True)
            nvalid = jnp.sum(valid.astype(jnp.float32))
            intra = jnp.sum(jnp.where(valid, diag, 0.0)) / jnp.maximum(
                nvalid, 1.0)
            off = jnp.where(eye | (~cen_valid).reshape(1, K), -inf, mean_pc)
            per_i_max = jnp.max(off, axis=1, keepdims=True)
            inter_min = jnp.min(jnp.where(valid, per_i_max, inf))
            return intra, inter_min

        intra_s, inter_s = terms(sd_ref[...], aux_ref[:, 2:3])
        intra_t, inter_t = terms(td_ref[...], aux_ref[:, 3:4])
        loss = intra_s + intra_t - 0.1 * (inter_s + inter_t)
        out_ref[...] = jnp.full((1, 1), loss, dtype=jnp.float32)


@functools.partial(jax.jit, static_argnames=("hb",))
def _run(sf, ssm, scon, tf, tsm, tcon, hb=32):
    B, C, h, w = sf.shape
    T = h // hb
    steps = B * T

    sf3 = sf.reshape(B * C, h, w)
    tf3 = tf.reshape(B * C, h, w)
    ssm3 = ssm.reshape(B * K, h, w)
    tsm3 = tsm.reshape(B * K, h, w)

    def fmap(p, i):
        return (i // T, i % T, 0)

    def smap(p, i):
        z = jnp.where(p == 0, 1, 0)
        return ((i // T) * z, (i % T) * z, 0)

    feat_spec = pl.BlockSpec((C, hb, w), fmap)
    sm_spec = pl.BlockSpec((K, hb, w), smap)
    w_spec = pl.BlockSpec((1, hb, w), smap)

    out = pl.pallas_call(
        functools.partial(_fused_kernel, T),
        grid=(2, steps),
        in_specs=[feat_spec, feat_spec, sm_spec, sm_spec, w_spec, w_spec],
        out_specs=pl.BlockSpec((1, 1), lambda p, i: (0, 0)),
        out_shape=jax.ShapeDtypeStruct((1, 1), jnp.float32),
        scratch_shapes=[
            pltpu.VMEM((C, K), jnp.float32),
            pltpu.VMEM((K, 4), jnp.float32),
            pltpu.VMEM((B * h, w), jnp.int32),
            pltpu.VMEM((B * h, w), jnp.int32),
            pltpu.VMEM((K, C), _BF),
            pltpu.VMEM((K, 1), jnp.float32),
            pltpu.VMEM((K, K), jnp.float32),
            pltpu.VMEM((K, K), jnp.float32),
        ],
    )(sf3, tf3, ssm3, tsm3, scon, tcon)
    return out.reshape(())


def kernel(source_feat, source_softmax, source_confidence,
           target_feat, target_softmax, target_confidence):
    return _run(source_feat, source_softmax, source_confidence,
                target_feat, target_softmax, target_confidence)
